# baseline (speedup 1.0000x reference)
import functools

import jax
import jax.numpy as jnp
from jax import lax
from jax.experimental import pallas as pl
from jax.experimental.pallas import tpu as pltpu
from jax.experimental.pallas import tpu_sc as plsc

B = 128
L = 50
EDIM = 128
KNUM = 100
KPAD = 128
MEM_SIZE = 262144
KEY_SIZE = 128
TOPK = 256
MARGIN = 0.1
CHUNK = 4096
PCHUNK = CHUNK // 2
NCHUNKS = MEM_SIZE // CHUNK
NEG_INF = -1e30


def _sc_gather(table, idx):
    v, d = table.shape
    n = idx.shape[0]
    info = plsc.get_sparse_core_info()
    nc, ns = info.num_cores, info.num_subcores
    nw = nc * ns
    b_per_w = n // nw
    mesh = plsc.VectorSubcoreMesh(core_axis_name="c", subcore_axis_name="s")

    @functools.partial(
        pl.kernel,
        mesh=mesh,
        out_type=jax.ShapeDtypeStruct((n, d), jnp.float32),
        scratch_types=[
            pltpu.VMEM((b_per_w,), jnp.int32),
            pltpu.VMEM((b_per_w, d), jnp.float32),
            pltpu.SemaphoreType.DMA,
        ],
    )
    def k(table_hbm, idx_hbm, out_hbm, idx_v, rows_v, sem):
        wid = lax.axis_index("s") * nc + lax.axis_index("c")
        base = wid * b_per_w
        pltpu.sync_copy(idx_hbm.at[pl.ds(base, b_per_w)], idx_v)
        pltpu.async_copy(table_hbm.at[idx_v], rows_v, sem).wait()
        pltpu.sync_copy(rows_v, out_hbm.at[pl.ds(base, b_per_w)])

    return k(table, idx)


def _scan_body(emb_ref, wcat_ref, bias_ref, pw_ref, pb_ref, y_ref,
               keys_ref, ve_ref, vo_ref, sims_out, gpos_ref, gneg_ref,
               q2_ref, prun_ref, nrun_ref):
    i = pl.program_id(0)

    @pl.when(i == 0)
    def _cnn():
        m = emb_ref[...]
        feats = []
        j = 0
        for w, t in ((3, L - 2), (4, L - 3), (5, L - 4)):
            c = None
            for dt in range(w):
                p = jnp.dot(m[dt * B:(dt + t) * B, :], wcat_ref[j],
                            preferred_element_type=jnp.float32)
                c = p if c is None else c + p
                j += 1
            cmax = jnp.max(c.reshape(t, B, KPAD), axis=0)
            widx = (3, 4, 5).index(w)
            feats.append(jnp.maximum(cmax + bias_ref[widx:widx + 1, :], 0.0))
        h = jnp.concatenate(feats, axis=1)
        q = jnp.dot(h, pw_ref[...], preferred_element_type=jnp.float32)
        q = q + pb_ref[...]
        nrm = jnp.sqrt(jnp.sum(q * q, axis=1, keepdims=True))
        q = q / (nrm + 1e-8)
        q2_ref[...] = jnp.zeros((2 * B, 2 * KEY_SIZE), jnp.float32)
        q2_ref[0:B, 0:KEY_SIZE] = q
        q2_ref[B:2 * B, KEY_SIZE:2 * KEY_SIZE] = q

    keys = keys_ref[...]
    sims = lax.dot_general(keys, q2_ref[...],
                           dimension_numbers=(((1,), (1,)), ((), ())),
                           preferred_element_type=jnp.float32)
    ve = jnp.transpose(ve_ref[0])
    vo = jnp.transpose(vo_ref[0])
    y = y_ref[...]
    pos = jnp.concatenate([ve == y, vo == y], axis=1)
    spos = jnp.where(pos, sims, NEG_INF)
    sneg = jnp.where(pos, NEG_INF, sims)

    @pl.when(i == 0)
    def _init():
        prun_ref[...] = spos
        nrun_ref[...] = sneg

    @pl.when(i > 0)
    def _acc():
        prun_ref[...] = jnp.maximum(prun_ref[...], spos)
        nrun_ref[...] = jnp.maximum(nrun_ref[...], sneg)

    sims_out[...] = sims.astype(jnp.bfloat16).reshape(1, PCHUNK, 2 * B)

    @pl.when(i == NCHUNKS - 1)
    def _reduce():
        mp = jnp.max(prun_ref[...], axis=0, keepdims=True)
        mn = jnp.max(nrun_ref[...], axis=0, keepdims=True)
        gpos_ref[...] = jnp.maximum(mp[:, :B], mp[:, B:])
        gneg_ref[...] = jnp.maximum(mn[:, :B], mn[:, B:])


def _count_body(sims_ref, gpos_ref, gneg_ref, acc_ref, loss_ref, cnt_ref):
    i = pl.program_id(0)

    lo = jnp.minimum(gpos_ref[...], gneg_ref[...])
    lo2 = jnp.concatenate([lo, lo], axis=1)
    s = sims_ref[0].astype(jnp.float32)
    inc = (s > lo2).astype(jnp.int32)

    @pl.when(i == 0)
    def _init():
        cnt_ref[...] = inc

    @pl.when(i > 0)
    def _acc():
        cnt_ref[...] += inc

    @pl.when(i == NCHUNKS - 1)
    def _final():
        gpos = gpos_ref[...]
        gneg = gneg_ref[...]
        c2 = jnp.sum(cnt_ref[...], axis=0, keepdims=True)
        has_both = (c2[:, :B] + c2[:, B:]) < TOPK
        lossv = jnp.where(has_both,
                          jnp.maximum(gneg - gpos + MARGIN, 0.0), 0.0)
        loss_ref[...] = jnp.reshape(jnp.sum(lossv) / B, (1, 1))
        acc_ref[...] = jnp.reshape(
            jnp.sum((gpos > gneg).astype(jnp.float32)) / B, (1, 1))


def kernel(x, y, embed_table, conv_w3, conv_b3, conv_w4, conv_b4,
           conv_w5, conv_b5, proj_W, proj_b, mem_keys, mem_values):
    idx = jnp.transpose(x).reshape(-1).astype(jnp.int32)
    emb = _sc_gather(embed_table, idx)

    taps = []
    for cw in (conv_w3, conv_w4, conv_w5):
        w = cw.shape[2]
        for dt in range(w):
            taps.append(jnp.pad(cw[:, :, dt].T, ((0, 0), (0, KPAD - KNUM))))
    wcat = jnp.stack(taps, axis=0)
    bias = jnp.stack([jnp.pad(b, (0, KPAD - KNUM))
                      for b in (conv_b3, conv_b4, conv_b5)], axis=0)
    pw = jnp.concatenate([
        jnp.pad(proj_W[i * KNUM:(i + 1) * KNUM, :], ((0, KPAD - KNUM), (0, 0)))
        for i in range(3)], axis=0)
    pb = proj_b.reshape(1, KEY_SIZE)
    y2 = y.reshape(1, B).astype(jnp.int32)
    keys2 = mem_keys.reshape(MEM_SIZE // 2, 2 * KEY_SIZE)
    mv = mem_values.astype(jnp.int32)
    ve3 = mv[0::2].reshape(NCHUNKS, 1, PCHUNK)
    vo3 = mv[1::2].reshape(NCHUNKS, 1, PCHUNK)

    sims_bf, gpos, gneg = pl.pallas_call(
        _scan_body,
        grid=(NCHUNKS,),
        in_specs=[
            pl.BlockSpec((L * B, EDIM), lambda i: (0, 0)),
            pl.BlockSpec((12, EDIM, KPAD), lambda i: (0, 0, 0)),
            pl.BlockSpec((3, KPAD), lambda i: (0, 0)),
            pl.BlockSpec((3 * KPAD, KEY_SIZE), lambda i: (0, 0)),
            pl.BlockSpec((1, KEY_SIZE), lambda i: (0, 0)),
            pl.BlockSpec((1, B), lambda i: (0, 0)),
            pl.BlockSpec((PCHUNK, 2 * KEY_SIZE), lambda i: (i, 0)),
            pl.BlockSpec((1, 1, PCHUNK), lambda i: (i, 0, 0)),
            pl.BlockSpec((1, 1, PCHUNK), lambda i: (i, 0, 0)),
        ],
        out_specs=[
            pl.BlockSpec((1, PCHUNK, 2 * B), lambda i: (i, 0, 0)),
            pl.BlockSpec((1, B), lambda i: (0, 0)),
            pl.BlockSpec((1, B), lambda i: (0, 0)),
        ],
        out_shape=[
            jax.ShapeDtypeStruct((NCHUNKS, PCHUNK, 2 * B), jnp.bfloat16),
            jax.ShapeDtypeStruct((1, B), jnp.float32),
            jax.ShapeDtypeStruct((1, B), jnp.float32),
        ],
        scratch_shapes=[pltpu.VMEM((2 * B, 2 * KEY_SIZE), jnp.float32),
                        pltpu.VMEM((PCHUNK, 2 * B), jnp.float32),
                        pltpu.VMEM((PCHUNK, 2 * B), jnp.float32)],
    )(emb, wcat, bias, pw, pb, y2, keys2, ve3, vo3)

    acc, loss = pl.pallas_call(
        _count_body,
        grid=(NCHUNKS,),
        in_specs=[
            pl.BlockSpec((1, PCHUNK, 2 * B), lambda i: (i, 0, 0)),
            pl.BlockSpec((1, B), lambda i: (0, 0)),
            pl.BlockSpec((1, B), lambda i: (0, 0)),
        ],
        out_specs=[
            pl.BlockSpec((1, 1), lambda i: (0, 0)),
            pl.BlockSpec((1, 1), lambda i: (0, 0)),
        ],
        out_shape=[
            jax.ShapeDtypeStruct((1, 1), jnp.float32),
            jax.ShapeDtypeStruct((1, 1), jnp.float32),
        ],
        scratch_shapes=[pltpu.VMEM((PCHUNK, 2 * B), jnp.int32)],
    )(sims_bf, gpos, gneg)

    return (acc[0, 0], loss[0, 0])


# speedup vs baseline: 2.0658x; 2.0658x over previous
import functools

import jax
import jax.numpy as jnp
from jax import lax
from jax.experimental import pallas as pl
from jax.experimental.pallas import tpu as pltpu
from jax.experimental.pallas import tpu_sc as plsc

B = 128
L = 50
EDIM = 128
KNUM = 100
KPAD = 128
MEM_SIZE = 262144
KEY_SIZE = 128
TOPK = 256
MARGIN = 0.1
CHUNK = 8192
NCHUNKS = MEM_SIZE // CHUNK
NEG_INF = -1e30


def _sc_gather(table, idx):
    v, d = table.shape
    n = idx.shape[0]
    info = plsc.get_sparse_core_info()
    nc, ns = info.num_cores, info.num_subcores
    nw = nc * ns
    b_per_w = n // nw
    mesh = plsc.VectorSubcoreMesh(core_axis_name="c", subcore_axis_name="s")

    @functools.partial(
        pl.kernel,
        mesh=mesh,
        out_type=jax.ShapeDtypeStruct((n, d), jnp.float32),
        scratch_types=[
            pltpu.VMEM((b_per_w,), jnp.int32),
            pltpu.VMEM((b_per_w, d), jnp.float32),
            pltpu.SemaphoreType.DMA,
        ],
    )
    def k(table_hbm, idx_hbm, out_hbm, idx_v, rows_v, sem):
        wid = lax.axis_index("s") * nc + lax.axis_index("c")
        base = wid * b_per_w
        pltpu.sync_copy(idx_hbm.at[pl.ds(base, b_per_w)], idx_v)
        pltpu.async_copy(table_hbm.at[idx_v], rows_v, sem).wait()
        pltpu.sync_copy(rows_v, out_hbm.at[pl.ds(base, b_per_w)])

    return k(table, idx)


def _scan_body(emb_ref, wcat_ref, bias_ref, pw_ref, pb_ref, y_ref,
               keys_ref, vals_ref, sims_out, gpos_ref, gneg_ref,
               q_ref, prun_ref, nrun_ref):
    i = pl.program_id(0)

    @pl.when(i == 0)
    def _cnn():
        m = emb_ref[...]
        feats = []
        j = 0
        for w, t in ((3, L - 2), (4, L - 3), (5, L - 4)):
            c = None
            for dt in range(w):
                p = jnp.dot(m[dt * B:(dt + t) * B, :], wcat_ref[j],
                            preferred_element_type=jnp.float32)
                c = p if c is None else c + p
                j += 1
            cmax = jnp.max(c.reshape(t, B, KPAD), axis=0)
            widx = (3, 4, 5).index(w)
            feats.append(jnp.maximum(cmax + bias_ref[widx:widx + 1, :], 0.0))
        h = jnp.concatenate(feats, axis=1)
        q = jnp.dot(h, pw_ref[...], preferred_element_type=jnp.float32)
        q = q + pb_ref[...]
        nrm = jnp.sqrt(jnp.sum(q * q, axis=1, keepdims=True))
        q_ref[...] = q / (nrm + 1e-8)

    keys = keys_ref[...]
    sims = lax.dot_general(keys, q_ref[...],
                           dimension_numbers=(((1,), (1,)), ((), ())),
                           preferred_element_type=jnp.float32)
    vals = jnp.transpose(vals_ref[0])
    pos = vals == y_ref[...]
    spos = jnp.where(pos, sims, NEG_INF)
    sneg = jnp.where(pos, NEG_INF, sims)

    @pl.when(i == 0)
    def _init():
        prun_ref[...] = spos
        nrun_ref[...] = sneg

    @pl.when(i > 0)
    def _acc():
        prun_ref[...] = jnp.maximum(prun_ref[...], spos)
        nrun_ref[...] = jnp.maximum(nrun_ref[...], sneg)

    sims_out[...] = sims.astype(jnp.bfloat16).reshape(1, CHUNK, B)

    @pl.when(i == NCHUNKS - 1)
    def _reduce():
        gpos_ref[...] = jnp.max(prun_ref[...], axis=0, keepdims=True)
        gneg_ref[...] = jnp.max(nrun_ref[...], axis=0, keepdims=True)


def _count_body(sims_ref, gpos_ref, gneg_ref, acc_ref, loss_ref, cnt_ref):
    i = pl.program_id(0)

    lo = jnp.minimum(gpos_ref[...], gneg_ref[...])
    s = sims_ref[0].astype(jnp.float32)
    inc = (s > lo).astype(jnp.int32)

    @pl.when(i == 0)
    def _init():
        cnt_ref[...] = inc

    @pl.when(i > 0)
    def _acc():
        cnt_ref[...] += inc

    @pl.when(i == NCHUNKS - 1)
    def _final():
        gpos = gpos_ref[...]
        gneg = gneg_ref[...]
        has_both = jnp.sum(cnt_ref[...], axis=0, keepdims=True) < TOPK
        lossv = jnp.where(has_both,
                          jnp.maximum(gneg - gpos + MARGIN, 0.0), 0.0)
        loss_ref[...] = jnp.reshape(jnp.sum(lossv) / B, (1, 1))
        acc_ref[...] = jnp.reshape(
            jnp.sum((gpos > gneg).astype(jnp.float32)) / B, (1, 1))


def kernel(x, y, embed_table, conv_w3, conv_b3, conv_w4, conv_b4,
           conv_w5, conv_b5, proj_W, proj_b, mem_keys, mem_values):
    idx = jnp.transpose(x).reshape(-1).astype(jnp.int32)
    emb = _sc_gather(embed_table, idx)

    taps = []
    for cw in (conv_w3, conv_w4, conv_w5):
        w = cw.shape[2]
        for dt in range(w):
            taps.append(jnp.pad(cw[:, :, dt].T, ((0, 0), (0, KPAD - KNUM))))
    wcat = jnp.stack(taps, axis=0)
    bias = jnp.stack([jnp.pad(b, (0, KPAD - KNUM))
                      for b in (conv_b3, conv_b4, conv_b5)], axis=0)
    pw = jnp.concatenate([
        jnp.pad(proj_W[i * KNUM:(i + 1) * KNUM, :], ((0, KPAD - KNUM), (0, 0)))
        for i in range(3)], axis=0)
    pb = proj_b.reshape(1, KEY_SIZE)
    y2 = y.reshape(1, B).astype(jnp.int32)
    vals3 = mem_values.astype(jnp.int32).reshape(NCHUNKS, 1, CHUNK)

    sims_bf, gpos, gneg = pl.pallas_call(
        _scan_body,
        grid=(NCHUNKS,),
        in_specs=[
            pl.BlockSpec((L * B, EDIM), lambda i: (0, 0)),
            pl.BlockSpec((12, EDIM, KPAD), lambda i: (0, 0, 0)),
            pl.BlockSpec((3, KPAD), lambda i: (0, 0)),
            pl.BlockSpec((3 * KPAD, KEY_SIZE), lambda i: (0, 0)),
            pl.BlockSpec((1, KEY_SIZE), lambda i: (0, 0)),
            pl.BlockSpec((1, B), lambda i: (0, 0)),
            pl.BlockSpec((CHUNK, KEY_SIZE), lambda i: (i, 0)),
            pl.BlockSpec((1, 1, CHUNK), lambda i: (i, 0, 0)),
        ],
        out_specs=[
            pl.BlockSpec((1, CHUNK, B), lambda i: (i, 0, 0)),
            pl.BlockSpec((1, B), lambda i: (0, 0)),
            pl.BlockSpec((1, B), lambda i: (0, 0)),
        ],
        out_shape=[
            jax.ShapeDtypeStruct((NCHUNKS, CHUNK, B), jnp.bfloat16),
            jax.ShapeDtypeStruct((1, B), jnp.float32),
            jax.ShapeDtypeStruct((1, B), jnp.float32),
        ],
        scratch_shapes=[pltpu.VMEM((B, KEY_SIZE), jnp.float32),
                        pltpu.VMEM((CHUNK, B), jnp.float32),
                        pltpu.VMEM((CHUNK, B), jnp.float32)],
    )(emb, wcat, bias, pw, pb, y2, mem_keys, vals3)

    acc, loss = pl.pallas_call(
        _count_body,
        grid=(NCHUNKS,),
        in_specs=[
            pl.BlockSpec((1, CHUNK, B), lambda i: (i, 0, 0)),
            pl.BlockSpec((1, B), lambda i: (0, 0)),
            pl.BlockSpec((1, B), lambda i: (0, 0)),
        ],
        out_specs=[
            pl.BlockSpec((1, 1), lambda i: (0, 0)),
            pl.BlockSpec((1, 1), lambda i: (0, 0)),
        ],
        out_shape=[
            jax.ShapeDtypeStruct((1, 1), jnp.float32),
            jax.ShapeDtypeStruct((1, 1), jnp.float32),
        ],
        scratch_shapes=[pltpu.VMEM((CHUNK, B), jnp.int32)],
    )(sims_bf, gpos, gneg)

    return (acc[0, 0], loss[0, 0])


# speedup vs baseline: 2.3580x; 1.1414x over previous
import functools

import jax
import jax.numpy as jnp
from jax import lax
from jax.experimental import pallas as pl
from jax.experimental.pallas import tpu as pltpu
from jax.experimental.pallas import tpu_sc as plsc

B = 128
L = 50
EDIM = 128
KNUM = 100
KPAD = 128
MEM_SIZE = 262144
KEY_SIZE = 128
TOPK = 256
MARGIN = 0.1
CHUNK = 8192
NCHUNKS = MEM_SIZE // CHUNK
NEG_INF = -1e30


def _sc_gather(table, idx):
    v, d = table.shape
    n = idx.shape[0]
    info = plsc.get_sparse_core_info()
    nc, ns = info.num_cores, info.num_subcores
    nw = nc * ns
    b_per_w = n // nw
    mesh = plsc.VectorSubcoreMesh(core_axis_name="c", subcore_axis_name="s")

    @functools.partial(
        pl.kernel,
        mesh=mesh,
        out_type=jax.ShapeDtypeStruct((n, d), jnp.float32),
        scratch_types=[
            pltpu.VMEM((b_per_w,), jnp.int32),
            pltpu.VMEM((b_per_w, d), jnp.float32),
            pltpu.SemaphoreType.DMA,
        ],
    )
    def k(table_hbm, idx_hbm, out_hbm, idx_v, rows_v, sem):
        wid = lax.axis_index("s") * nc + lax.axis_index("c")
        base = wid * b_per_w
        pltpu.sync_copy(idx_hbm.at[pl.ds(base, b_per_w)], idx_v)
        pltpu.async_copy(table_hbm.at[idx_v], rows_v, sem).wait()
        pltpu.sync_copy(rows_v, out_hbm.at[pl.ds(base, b_per_w)])

    return k(table, idx)


def _scan_body(emb_ref, wcat_ref, bias_ref, pw_ref, pb_ref, y_ref,
               keys_ref, vals_ref, sims_out, gpos_ref, gneg_ref, q_ref):
    i = pl.program_id(0)

    @pl.when(i == 0)
    def _cnn():
        m = emb_ref[...]
        feats = []
        j = 0
        for w, t in ((3, L - 2), (4, L - 3), (5, L - 4)):
            c = None
            for dt in range(w):
                p = jnp.dot(m[dt * B:(dt + t) * B, :], wcat_ref[j],
                            preferred_element_type=jnp.float32)
                c = p if c is None else c + p
                j += 1
            cmax = jnp.max(c.reshape(t, B, KPAD), axis=0)
            widx = (3, 4, 5).index(w)
            feats.append(jnp.maximum(cmax + bias_ref[widx:widx + 1, :], 0.0))
        h = jnp.concatenate(feats, axis=1)
        q = jnp.dot(h, pw_ref[...], preferred_element_type=jnp.float32)
        q = q + pb_ref[...]
        nrm = jnp.sqrt(jnp.sum(q * q, axis=1, keepdims=True))
        q_ref[...] = q / (nrm + 1e-8)

    keys = keys_ref[...]
    sims = lax.dot_general(keys, q_ref[...],
                           dimension_numbers=(((1,), (1,)), ((), ())),
                           preferred_element_type=jnp.float32)
    vals = jnp.transpose(vals_ref[0])
    pos = vals == y_ref[...]
    pmax = jnp.max(jnp.where(pos, sims, NEG_INF), axis=0, keepdims=True)
    nmax = jnp.max(jnp.where(pos, NEG_INF, sims), axis=0, keepdims=True)

    @pl.when(i == 0)
    def _init():
        gpos_ref[...] = pmax
        gneg_ref[...] = nmax

    @pl.when(i > 0)
    def _acc():
        gpos_ref[...] = jnp.maximum(gpos_ref[...], pmax)
        gneg_ref[...] = jnp.maximum(gneg_ref[...], nmax)

    sims_out[...] = sims.astype(jnp.bfloat16).reshape(1, CHUNK, B)


def _count_body(sims_ref, gpos_ref, gneg_ref, acc_ref, loss_ref, cnt_ref):
    i = pl.program_id(0)

    lo = jnp.minimum(gpos_ref[...], gneg_ref[...])
    s = sims_ref[0].astype(jnp.float32)
    inc = jnp.sum((s > lo).astype(jnp.int32), axis=0, keepdims=True)

    @pl.when(i == 0)
    def _init():
        cnt_ref[...] = inc

    @pl.when(i > 0)
    def _acc():
        cnt_ref[...] += inc

    @pl.when(i == NCHUNKS - 1)
    def _final():
        gpos = gpos_ref[...]
        gneg = gneg_ref[...]
        has_both = cnt_ref[...] < TOPK
        lossv = jnp.where(has_both,
                          jnp.maximum(gneg - gpos + MARGIN, 0.0), 0.0)
        loss_ref[...] = jnp.reshape(jnp.sum(lossv) / B, (1, 1))
        acc_ref[...] = jnp.reshape(
            jnp.sum((gpos > gneg).astype(jnp.float32)) / B, (1, 1))


def kernel(x, y, embed_table, conv_w3, conv_b3, conv_w4, conv_b4,
           conv_w5, conv_b5, proj_W, proj_b, mem_keys, mem_values):
    idx = jnp.transpose(x).reshape(-1).astype(jnp.int32)
    emb = _sc_gather(embed_table, idx)

    taps = []
    for cw in (conv_w3, conv_w4, conv_w5):
        w = cw.shape[2]
        for dt in range(w):
            taps.append(jnp.pad(cw[:, :, dt].T, ((0, 0), (0, KPAD - KNUM))))
    wcat = jnp.stack(taps, axis=0)
    bias = jnp.stack([jnp.pad(b, (0, KPAD - KNUM))
                      for b in (conv_b3, conv_b4, conv_b5)], axis=0)
    pw = jnp.concatenate([
        jnp.pad(proj_W[i * KNUM:(i + 1) * KNUM, :], ((0, KPAD - KNUM), (0, 0)))
        for i in range(3)], axis=0)
    pb = proj_b.reshape(1, KEY_SIZE)
    y2 = y.reshape(1, B).astype(jnp.int32)
    vals3 = mem_values.astype(jnp.int32).reshape(NCHUNKS, 1, CHUNK)

    sims_bf, gpos, gneg = pl.pallas_call(
        _scan_body,
        grid=(NCHUNKS,),
        in_specs=[
            pl.BlockSpec((L * B, EDIM), lambda i: (0, 0)),
            pl.BlockSpec((12, EDIM, KPAD), lambda i: (0, 0, 0)),
            pl.BlockSpec((3, KPAD), lambda i: (0, 0)),
            pl.BlockSpec((3 * KPAD, KEY_SIZE), lambda i: (0, 0)),
            pl.BlockSpec((1, KEY_SIZE), lambda i: (0, 0)),
            pl.BlockSpec((1, B), lambda i: (0, 0)),
            pl.BlockSpec((CHUNK, KEY_SIZE), lambda i: (i, 0)),
            pl.BlockSpec((1, 1, CHUNK), lambda i: (i, 0, 0)),
        ],
        out_specs=[
            pl.BlockSpec((1, CHUNK, B), lambda i: (i, 0, 0)),
            pl.BlockSpec((1, B), lambda i: (0, 0)),
            pl.BlockSpec((1, B), lambda i: (0, 0)),
        ],
        out_shape=[
            jax.ShapeDtypeStruct((NCHUNKS, CHUNK, B), jnp.bfloat16),
            jax.ShapeDtypeStruct((1, B), jnp.float32),
            jax.ShapeDtypeStruct((1, B), jnp.float32),
        ],
        scratch_shapes=[pltpu.VMEM((B, KEY_SIZE), jnp.float32)],
    )(emb, wcat, bias, pw, pb, y2, mem_keys, vals3)

    acc, loss = pl.pallas_call(
        _count_body,
        grid=(NCHUNKS,),
        in_specs=[
            pl.BlockSpec((1, CHUNK, B), lambda i: (i, 0, 0)),
            pl.BlockSpec((1, B), lambda i: (0, 0)),
            pl.BlockSpec((1, B), lambda i: (0, 0)),
        ],
        out_specs=[
            pl.BlockSpec((1, 1), lambda i: (0, 0)),
            pl.BlockSpec((1, 1), lambda i: (0, 0)),
        ],
        out_shape=[
            jax.ShapeDtypeStruct((1, 1), jnp.float32),
            jax.ShapeDtypeStruct((1, 1), jnp.float32),
        ],
        scratch_shapes=[pltpu.VMEM((1, B), jnp.int32)],
    )(sims_bf, gpos, gneg)

    return (acc[0, 0], loss[0, 0])
